# baseline (speedup 1.0000x reference)
import functools

import jax
import jax.numpy as jnp
from jax import lax
from jax.experimental import pallas as pl
from jax.experimental.pallas import tpu as pltpu
from jax.experimental.pallas import tpu_sc as plsc

NU = 25000
NI = 25000
D = 64
DP = 128
E = 800000
HALF = 25088
NP = 2 * HALF
HROW = HALF // 2
TRASH = 12500
NTILE = 16
EPT = E // NTILE
SUP = 2000
NSUP = EPT // SUP
CH = 80
NCH = SUP // CH
TPT = HROW // NTILE

_MESH = plsc.VectorSubcoreMesh(core_axis_name="c", subcore_axis_name="s")


def _spmm_body(tab, colh, rowh, valh, out,
               colb, rowb, valb, colx, dstx, rowsb, acc, sem):
    cid = lax.axis_index("c")
    sid = lax.axis_index("s")

    zv = jnp.zeros((16,), jnp.float32)

    def zb(r, carry):
        for c in range(DP // 16):
            rowsb[r, pl.ds(c * 16, 16)] = zv
        return carry

    lax.fori_loop(0, CH, zb, 0)
    tbase = sid * TPT
    for k in range(TPT // CH):
        pltpu.sync_copy(rowsb, acc.at[pl.ds(tbase + k * CH, CH)])
    if TPT % CH:
        pltpu.sync_copy(rowsb.at[pl.ds(0, TPT % CH)],
                        acc.at[pl.ds(tbase + (TPT // CH) * CH, TPT % CH)])
    plsc.subcore_barrier()

    rbase = cid * HALF

    def sup_body(s, carry):
        eb = sid * EPT + s * SUP
        pltpu.sync_copy(colh.at[pl.ds(eb, SUP)], colb)
        pltpu.sync_copy(rowh.at[pl.ds(eb, SUP)], rowb)
        pltpu.sync_copy(valh.at[pl.ds(eb, SUP)], valb)

        def ch_body(j, c2):
            off = j * CH
            for k in range(CH // 16):
                sl = pl.ds(off + k * 16, 16)
                dl = pl.ds(k * 16, 16)
                c16 = colb[sl]
                cpad = c16 + jnp.where(c16 >= NU, HALF - NU, 0)
                r16 = rowb[sl]
                rpad = r16 + jnp.where(r16 >= NU, HALF - NU, 0)
                loc = rpad - rbase
                ok = (loc >= 0) & (loc < HALF)
                locs = jnp.where(ok, loc, 2 * TRASH)
                colx[dl] = cpad + (locs & 1) * NP
                dstx[0, dl] = locs >> 1
            pltpu.async_copy(tab.at[colx], rowsb, sem).wait()

            for g in range(CH // 16):
                sl16 = pl.ds(off + g * 16, 16)
                vv = valb[sl16]
                r16 = rowb[sl16]
                rpad = r16 + jnp.where(r16 >= NU, HALF - NU, 0)
                loc = rpad - rbase
                ok = (loc >= 0) & (loc < HALF)
                pv = (jnp.where(ok, loc, 2 * TRASH) & 1) * D
                for e in range(16):
                    v = vv[e]
                    p = pv[e]
                    er = g * 16 + e
                    for c in range(D // 16):
                        s2 = pl.ds(p + c * 16, 16)
                        rowsb[er, s2] = rowsb[er, s2] * v
            pltpu.sync_copy(rowsb, acc.at[dstx.at[0]], add=True)
            return c2

        lax.fori_loop(0, NCH, ch_body, 0)
        return carry

    lax.fori_loop(0, NSUP, sup_body, 0)
    plsc.subcore_barrier()

    pltpu.sync_copy(acc.at[pl.ds(tbase, TPT)],
                    out.at[pl.ds(cid * HROW + tbase, TPT)])


_spmm = functools.partial(
    pl.kernel,
    mesh=_MESH,
    out_type=jax.ShapeDtypeStruct((NP // 2, DP), jnp.float32),
    scratch_types=[
        pltpu.VMEM((SUP,), jnp.int32),
        pltpu.VMEM((SUP,), jnp.int32),
        pltpu.VMEM((SUP,), jnp.float32),
        pltpu.VMEM((CH,), jnp.int32),
        pltpu.VMEM((1, CH), jnp.int32),
        pltpu.VMEM((CH, DP), jnp.float32),
        pltpu.VMEM_SHARED((HROW, DP), jnp.float32),
        pltpu.SemaphoreType.DMA,
    ],
)(_spmm_body)


def _gather_body(ue, ie, n1, n2, ui, pi, ni, pio, nio,
                 ub, pb, nb, u1, p1, n1o, u2, p2, n2o,
                 idxv, buf, sem):
    wid = lax.axis_index("s") * 2 + lax.axis_index("c")
    base = wid * 128
    for tab, idx, outr in ((ue, ui, ub), (ie, pi, pb), (ie, ni, nb),
                           (n1, ui, u1), (n1, pio, p1), (n1, nio, n1o),
                           (n2, ui, u2), (n2, pio, p2), (n2, nio, n2o)):
        pltpu.sync_copy(idx.at[pl.ds(base, 128)], idxv)
        pltpu.async_copy(tab.at[idxv], buf, sem).wait()
        pltpu.sync_copy(buf, outr.at[pl.ds(base, 128)])


_gather = functools.partial(
    pl.kernel,
    mesh=_MESH,
    out_type=tuple(jax.ShapeDtypeStruct((4096, DP), jnp.float32)
                   for _ in range(9)),
    scratch_types=[
        pltpu.VMEM((128,), jnp.int32),
        pltpu.VMEM((128, DP), jnp.float32),
        pltpu.SemaphoreType.DMA,
    ],
)(_gather_body)


def _dense_body(side_ref, ego_ref, w1_ref, b1_ref, w2_ref, b2_ref,
                ego_out, tabl_out, tabr_out, norm_out):
    s = side_ref[...]
    e = ego_ref[...]
    h = (jnp.dot(s, w1_ref[...], preferred_element_type=jnp.float32)
         + b1_ref[...]
         + jnp.dot(e * s, w2_ref[...], preferred_element_type=jnp.float32)
         + b2_ref[...])
    h = jnp.where(h >= 0, h, 0.2 * h)
    nrm = jnp.sqrt(jnp.sum(h * h, axis=1, keepdims=True))
    z = jnp.zeros_like(h)
    ego_out[...] = h
    tabl_out[...] = jnp.concatenate([h, z], axis=1)
    tabr_out[...] = jnp.concatenate([z, h], axis=1)
    norm_out[...] = jnp.concatenate([h / jnp.maximum(nrm, 1e-12), z], axis=1)


_dense = pl.pallas_call(
    _dense_body,
    grid=(NP // 512,),
    in_specs=[
        pl.BlockSpec((512, D), lambda i: (i, 0)),
        pl.BlockSpec((512, D), lambda i: (i, 0)),
        pl.BlockSpec((D, D), lambda i: (0, 0)),
        pl.BlockSpec((1, D), lambda i: (0, 0)),
        pl.BlockSpec((D, D), lambda i: (0, 0)),
        pl.BlockSpec((1, D), lambda i: (0, 0)),
    ],
    out_specs=[pl.BlockSpec((512, D), lambda i: (i, 0)),
               pl.BlockSpec((512, DP), lambda i: (i, 0)),
               pl.BlockSpec((512, DP), lambda i: (i, 0)),
               pl.BlockSpec((512, DP), lambda i: (i, 0))],
    out_shape=[jax.ShapeDtypeStruct((NP, D), jnp.float32),
               jax.ShapeDtypeStruct((NP, DP), jnp.float32),
               jax.ShapeDtypeStruct((NP, DP), jnp.float32),
               jax.ShapeDtypeStruct((NP, DP), jnp.float32)],
)


def kernel(users, pos_items, neg_items, edge_index, edge_vals,
           user_emb, item_emb,
           W1_0, b1_0, W2_0, b2_0, W1_1, b1_1, W2_1, b2_1):
    row = edge_index[0].astype(jnp.int32)
    col = edge_index[1].astype(jnp.int32)
    vals = edge_vals.astype(jnp.float32)
    ui = users.astype(jnp.int32)
    pi = pos_items.astype(jnp.int32)
    ni = neg_items.astype(jnp.int32)
    pio = pi + HALF
    nio = ni + HALF

    ego = (jnp.zeros((NP, D), jnp.float32)
           .at[:NU].set(user_emb)
           .at[HALF:HALF + NI].set(item_emb))
    tabl = jnp.concatenate([ego, jnp.zeros((NP, D), jnp.float32)], axis=1)
    tabr = jnp.concatenate([jnp.zeros((NP, D), jnp.float32), ego], axis=1)

    ue128 = jnp.concatenate(
        [user_emb, jnp.zeros((NU, D), jnp.float32)], axis=1)
    ie128 = jnp.concatenate(
        [item_emb, jnp.zeros((NI, D), jnp.float32)], axis=1)

    norms = []
    for W1, b1, W2, b2 in ((W1_0, b1_0, W2_0, b2_0),
                           (W1_1, b1_1, W2_1, b2_1)):
        tab = jnp.concatenate([tabl, tabr], axis=0)
        side = _spmm(tab, col, row, vals).reshape(NP, D)
        ego, tabl, tabr, nrm = _dense(side, ego, W1, b1, W2, b2)
        norms.append(nrm)

    (ub, pb, nb, u1, p1, n1o, u2, p2, n2o) = _gather(
        ue128, ie128, norms[0], norms[1], ui, pi, ni, pio, nio)

    u_out = jnp.concatenate([ub[:, :D], u1[:, :D], u2[:, :D]], axis=1)
    pos_out = jnp.concatenate([pb[:, :D], p1[:, :D], p2[:, :D]], axis=1)
    neg_out = jnp.concatenate([nb[:, :D], n1o[:, :D], n2o[:, :D]], axis=1)
    return (u_out, pos_out, neg_out)


# speedup vs baseline: 1.0157x; 1.0157x over previous
import functools

import jax
import jax.numpy as jnp
from jax import lax
from jax.experimental import pallas as pl
from jax.experimental.pallas import tpu as pltpu
from jax.experimental.pallas import tpu_sc as plsc

NU = 25000
NI = 25000
D = 64
DP = 128
E = 800000
HALF = 25088
NP = 2 * HALF
HROW = HALF // 2
TRASH = 12500
NTILE = 16
EPT = E // NTILE
SUP = 2000
NSUP = EPT // SUP
CH = 80
NCH = SUP // CH
TPT = HROW // NTILE

_MESH = plsc.VectorSubcoreMesh(core_axis_name="c", subcore_axis_name="s")


def _spmm_body(tab, colh, rowh, valh, out,
               colb, rowb, valb, colx, dstx, rowsb,
               colx2, dstx2, rowsb2, acc, sem, sem2):
    cid = lax.axis_index("c")
    sid = lax.axis_index("s")

    zv = jnp.zeros((16,), jnp.float32)

    def zb(r, carry):
        for c in range(DP // 16):
            rowsb[r, pl.ds(c * 16, 16)] = zv
        return carry

    lax.fori_loop(0, CH, zb, 0)
    tbase = sid * TPT
    for k in range(TPT // CH):
        pltpu.sync_copy(rowsb, acc.at[pl.ds(tbase + k * CH, CH)])
    if TPT % CH:
        pltpu.sync_copy(rowsb.at[pl.ds(0, TPT % CH)],
                        acc.at[pl.ds(tbase + (TPT // CH) * CH, TPT % CH)])
    plsc.subcore_barrier()

    rbase = cid * HALF

    def sup_body(s, carry):
        eb = sid * EPT + s * SUP
        pltpu.sync_copy(colh.at[pl.ds(eb, SUP)], colb)
        pltpu.sync_copy(rowh.at[pl.ds(eb, SUP)], rowb)
        pltpu.sync_copy(valh.at[pl.ds(eb, SUP)], valb)

        def idx_chunk(j, colxr, dstxr):
            off = j * CH
            for k in range(CH // 16):
                sl = pl.ds(off + k * 16, 16)
                dl = pl.ds(k * 16, 16)
                c16 = colb[sl]
                cpad = c16 + jnp.where(c16 >= NU, HALF - NU, 0)
                r16 = rowb[sl]
                rpad = r16 + jnp.where(r16 >= NU, HALF - NU, 0)
                loc = rpad - rbase
                ok = (loc >= 0) & (loc < HALF)
                locs = jnp.where(ok, loc, 2 * TRASH)
                colxr[dl] = cpad + (locs & 1) * NP
                dstxr[0, dl] = locs >> 1

        def mul_scatter(j, rowsbr, dstxr):
            off = j * CH
            for g in range(CH // 16):
                vv = valb[pl.ds(off + g * 16, 16)]
                for e in range(16):
                    v = vv[e]
                    er = g * 16 + e
                    for c in range(DP // 16):
                        s2 = pl.ds(c * 16, 16)
                        rowsbr[er, s2] = rowsbr[er, s2] * v
            pltpu.sync_copy(rowsbr, acc.at[dstxr.at[0]], add=True)

        def pair_body(t, c2):
            j0 = 2 * t
            idx_chunk(j0, colx, dstx)
            h0 = pltpu.async_copy(tab.at[colx], rowsb, sem)
            idx_chunk(j0 + 1, colx2, dstx2)
            h1 = pltpu.async_copy(tab.at[colx2], rowsb2, sem2)
            h0.wait()
            mul_scatter(j0, rowsb, dstx)
            h1.wait()
            mul_scatter(j0 + 1, rowsb2, dstx2)
            return c2

        lax.fori_loop(0, NCH // 2, pair_body, 0)
        idx_chunk(NCH - 1, colx, dstx)
        pltpu.async_copy(tab.at[colx], rowsb, sem).wait()
        mul_scatter(NCH - 1, rowsb, dstx)
        return carry

    lax.fori_loop(0, NSUP, sup_body, 0)
    plsc.subcore_barrier()

    pltpu.sync_copy(acc.at[pl.ds(tbase, TPT)],
                    out.at[pl.ds(cid * HROW + tbase, TPT)])


_spmm = functools.partial(
    pl.kernel,
    mesh=_MESH,
    out_type=jax.ShapeDtypeStruct((NP // 2, DP), jnp.float32),
    scratch_types=[
        pltpu.VMEM((SUP,), jnp.int32),
        pltpu.VMEM((SUP,), jnp.int32),
        pltpu.VMEM((SUP,), jnp.float32),
        pltpu.VMEM((CH,), jnp.int32),
        pltpu.VMEM((1, CH), jnp.int32),
        pltpu.VMEM((CH, DP), jnp.float32),
        pltpu.VMEM((CH,), jnp.int32),
        pltpu.VMEM((1, CH), jnp.int32),
        pltpu.VMEM((CH, DP), jnp.float32),
        pltpu.VMEM_SHARED((HROW, DP), jnp.float32),
        pltpu.SemaphoreType.DMA,
        pltpu.SemaphoreType.DMA,
    ],
)(_spmm_body)


def _gather_body(ue, ie, n1, n2, ui, pi, ni, pio, nio,
                 ub, pb, nb, u1, p1, n1o, u2, p2, n2o,
                 idxv, buf, sem):
    wid = lax.axis_index("s") * 2 + lax.axis_index("c")
    base = wid * 128
    for tab, idx, outr in ((ue, ui, ub), (ie, pi, pb), (ie, ni, nb),
                           (n1, ui, u1), (n1, pio, p1), (n1, nio, n1o),
                           (n2, ui, u2), (n2, pio, p2), (n2, nio, n2o)):
        pltpu.sync_copy(idx.at[pl.ds(base, 128)], idxv)
        pltpu.async_copy(tab.at[idxv], buf, sem).wait()
        pltpu.sync_copy(buf, outr.at[pl.ds(base, 128)])


_gather = functools.partial(
    pl.kernel,
    mesh=_MESH,
    out_type=tuple(jax.ShapeDtypeStruct((4096, DP), jnp.float32)
                   for _ in range(9)),
    scratch_types=[
        pltpu.VMEM((128,), jnp.int32),
        pltpu.VMEM((128, DP), jnp.float32),
        pltpu.SemaphoreType.DMA,
    ],
)(_gather_body)


def _dense_body(side_ref, ego_ref, w1_ref, b1_ref, w2_ref, b2_ref,
                ego_out, tabl_out, tabr_out, norm_out):
    s = side_ref[...]
    e = ego_ref[...]
    h = (jnp.dot(s, w1_ref[...], preferred_element_type=jnp.float32)
         + b1_ref[...]
         + jnp.dot(e * s, w2_ref[...], preferred_element_type=jnp.float32)
         + b2_ref[...])
    h = jnp.where(h >= 0, h, 0.2 * h)
    nrm = jnp.sqrt(jnp.sum(h * h, axis=1, keepdims=True))
    z = jnp.zeros_like(h)
    ego_out[...] = h
    tabl_out[...] = jnp.concatenate([h, z], axis=1)
    tabr_out[...] = jnp.concatenate([z, h], axis=1)
    norm_out[...] = jnp.concatenate([h / jnp.maximum(nrm, 1e-12), z], axis=1)


_dense = pl.pallas_call(
    _dense_body,
    grid=(NP // 512,),
    in_specs=[
        pl.BlockSpec((512, D), lambda i: (i, 0)),
        pl.BlockSpec((512, D), lambda i: (i, 0)),
        pl.BlockSpec((D, D), lambda i: (0, 0)),
        pl.BlockSpec((1, D), lambda i: (0, 0)),
        pl.BlockSpec((D, D), lambda i: (0, 0)),
        pl.BlockSpec((1, D), lambda i: (0, 0)),
    ],
    out_specs=[pl.BlockSpec((512, D), lambda i: (i, 0)),
               pl.BlockSpec((512, DP), lambda i: (i, 0)),
               pl.BlockSpec((512, DP), lambda i: (i, 0)),
               pl.BlockSpec((512, DP), lambda i: (i, 0))],
    out_shape=[jax.ShapeDtypeStruct((NP, D), jnp.float32),
               jax.ShapeDtypeStruct((NP, DP), jnp.float32),
               jax.ShapeDtypeStruct((NP, DP), jnp.float32),
               jax.ShapeDtypeStruct((NP, DP), jnp.float32)],
)


def kernel(users, pos_items, neg_items, edge_index, edge_vals,
           user_emb, item_emb,
           W1_0, b1_0, W2_0, b2_0, W1_1, b1_1, W2_1, b2_1):
    row = edge_index[0].astype(jnp.int32)
    col = edge_index[1].astype(jnp.int32)
    vals = edge_vals.astype(jnp.float32)
    ui = users.astype(jnp.int32)
    pi = pos_items.astype(jnp.int32)
    ni = neg_items.astype(jnp.int32)
    pio = pi + HALF
    nio = ni + HALF

    ego = (jnp.zeros((NP, D), jnp.float32)
           .at[:NU].set(user_emb)
           .at[HALF:HALF + NI].set(item_emb))
    tabl = jnp.concatenate([ego, jnp.zeros((NP, D), jnp.float32)], axis=1)
    tabr = jnp.concatenate([jnp.zeros((NP, D), jnp.float32), ego], axis=1)

    ue128 = jnp.concatenate(
        [user_emb, jnp.zeros((NU, D), jnp.float32)], axis=1)
    ie128 = jnp.concatenate(
        [item_emb, jnp.zeros((NI, D), jnp.float32)], axis=1)

    norms = []
    for W1, b1, W2, b2 in ((W1_0, b1_0, W2_0, b2_0),
                           (W1_1, b1_1, W2_1, b2_1)):
        tab = jnp.concatenate([tabl, tabr], axis=0)
        side = _spmm(tab, col, row, vals).reshape(NP, D)
        ego, tabl, tabr, nrm = _dense(side, ego, W1, b1, W2, b2)
        norms.append(nrm)

    (ub, pb, nb, u1, p1, n1o, u2, p2, n2o) = _gather(
        ue128, ie128, norms[0], norms[1], ui, pi, ni, pio, nio)

    u_out = jnp.concatenate([ub[:, :D], u1[:, :D], u2[:, :D]], axis=1)
    pos_out = jnp.concatenate([pb[:, :D], p1[:, :D], p2[:, :D]], axis=1)
    neg_out = jnp.concatenate([nb[:, :D], n1o[:, :D], n2o[:, :D]], axis=1)
    return (u_out, pos_out, neg_out)


# speedup vs baseline: 1.0759x; 1.0593x over previous
import functools

import jax
import jax.numpy as jnp
from jax import lax
from jax.experimental import pallas as pl
from jax.experimental.pallas import tpu as pltpu
from jax.experimental.pallas import tpu_sc as plsc

NU = 25000
NI = 25000
D = 64
DP = 128
E = 800000
HALF = 25088
NP = 2 * HALF
HROW = HALF // 2
TRASH = 12500
NTILE = 16
EPT = E // NTILE
SUP = 2000
NSUP = EPT // SUP
CH = 80
NCH = SUP // CH
TPT = HROW // NTILE

_MESH = plsc.VectorSubcoreMesh(core_axis_name="c", subcore_axis_name="s")


def _spmm_body(tab, colh, rowh, valh, out,
               colb, rowb, valb, colx, dstx, rowsb, acc, sem):
    cid = lax.axis_index("c")
    sid = lax.axis_index("s")

    zv = jnp.zeros((16,), jnp.float32)

    def zb(r, carry):
        for c in range(DP // 16):
            rowsb[r, pl.ds(c * 16, 16)] = zv
        return carry

    lax.fori_loop(0, CH, zb, 0)
    tbase = sid * TPT
    for k in range(TPT // CH):
        pltpu.sync_copy(rowsb, acc.at[pl.ds(tbase + k * CH, CH)])
    if TPT % CH:
        pltpu.sync_copy(rowsb.at[pl.ds(0, TPT % CH)],
                        acc.at[pl.ds(tbase + (TPT // CH) * CH, TPT % CH)])
    plsc.subcore_barrier()

    rbase = cid * HALF

    def sup_body(s, carry):
        eb = sid * EPT + s * SUP
        pltpu.sync_copy(colh.at[pl.ds(eb, SUP)], colb)
        pltpu.sync_copy(rowh.at[pl.ds(eb, SUP)], rowb)
        pltpu.sync_copy(valh.at[pl.ds(eb, SUP)], valb)

        def ch_body(j, c2):
            off = j * CH
            for k in range(CH // 16):
                sl = pl.ds(off + k * 16, 16)
                dl = pl.ds(k * 16, 16)
                c16 = colb[sl]
                cpad = c16 + jnp.where(c16 >= NU, HALF - NU, 0)
                r16 = rowb[sl]
                rpad = r16 + jnp.where(r16 >= NU, HALF - NU, 0)
                loc = rpad - rbase
                ok = (loc >= 0) & (loc < HALF)
                locs = jnp.where(ok, loc, 2 * TRASH)
                colx[dl] = cpad + (locs & 1) * NP
                dstx[0, dl] = locs >> 1
            pltpu.async_copy(tab.at[colx], rowsb, sem).wait()

            for g in range(CH // 16):
                vv = valb[pl.ds(off + g * 16, 16)]
                for e in range(16):
                    v = vv[e]
                    er = g * 16 + e
                    for c in range(DP // 16):
                        s2 = pl.ds(c * 16, 16)
                        rowsb[er, s2] = rowsb[er, s2] * v
            pltpu.sync_copy(rowsb, acc.at[dstx.at[0]], add=True)
            return c2

        lax.fori_loop(0, NCH, ch_body, 0)
        return carry

    lax.fori_loop(0, NSUP, sup_body, 0)
    plsc.subcore_barrier()

    pltpu.sync_copy(acc.at[pl.ds(tbase, TPT)],
                    out.at[pl.ds(cid * HROW + tbase, TPT)])


_spmm = functools.partial(
    pl.kernel,
    mesh=_MESH,
    out_type=jax.ShapeDtypeStruct((NP // 2, DP), jnp.float32),
    scratch_types=[
        pltpu.VMEM((SUP,), jnp.int32),
        pltpu.VMEM((SUP,), jnp.int32),
        pltpu.VMEM((SUP,), jnp.float32),
        pltpu.VMEM((CH,), jnp.int32),
        pltpu.VMEM((1, CH), jnp.int32),
        pltpu.VMEM((CH, DP), jnp.float32),
        pltpu.VMEM_SHARED((HROW, DP), jnp.float32),
        pltpu.SemaphoreType.DMA,
    ],
)(_spmm_body)


def _gather_body(ue, ie, n1, n2, ui, pi, ni, pio, nio,
                 ub, pb, nb, u1, p1, n1o, u2, p2, n2o,
                 idxv, buf, sem):
    wid = lax.axis_index("s") * 2 + lax.axis_index("c")
    base = wid * 128
    for tab, idx, outr in ((ue, ui, ub), (ie, pi, pb), (ie, ni, nb),
                           (n1, ui, u1), (n1, pio, p1), (n1, nio, n1o),
                           (n2, ui, u2), (n2, pio, p2), (n2, nio, n2o)):
        pltpu.sync_copy(idx.at[pl.ds(base, 128)], idxv)
        pltpu.async_copy(tab.at[idxv], buf, sem).wait()
        pltpu.sync_copy(buf, outr.at[pl.ds(base, 128)])


_gather = functools.partial(
    pl.kernel,
    mesh=_MESH,
    out_type=tuple(jax.ShapeDtypeStruct((4096, DP), jnp.float32)
                   for _ in range(9)),
    scratch_types=[
        pltpu.VMEM((128,), jnp.int32),
        pltpu.VMEM((128, DP), jnp.float32),
        pltpu.SemaphoreType.DMA,
    ],
)(_gather_body)


def _dense_body(side_ref, ego_ref, w1_ref, b1_ref, w2_ref, b2_ref,
                ego_out, tabl_out, tabr_out, norm_out):
    s = side_ref[...]
    e = ego_ref[...]
    h = (jnp.dot(s, w1_ref[...], preferred_element_type=jnp.float32)
         + b1_ref[...]
         + jnp.dot(e * s, w2_ref[...], preferred_element_type=jnp.float32)
         + b2_ref[...])
    h = jnp.where(h >= 0, h, 0.2 * h)
    nrm = jnp.sqrt(jnp.sum(h * h, axis=1, keepdims=True))
    z = jnp.zeros_like(h)
    ego_out[...] = h
    tabl_out[...] = jnp.concatenate([h, z], axis=1)
    tabr_out[...] = jnp.concatenate([z, h], axis=1)
    norm_out[...] = jnp.concatenate([h / jnp.maximum(nrm, 1e-12), z], axis=1)


_dense = pl.pallas_call(
    _dense_body,
    grid=(NP // 512,),
    in_specs=[
        pl.BlockSpec((512, D), lambda i: (i, 0)),
        pl.BlockSpec((512, D), lambda i: (i, 0)),
        pl.BlockSpec((D, D), lambda i: (0, 0)),
        pl.BlockSpec((1, D), lambda i: (0, 0)),
        pl.BlockSpec((D, D), lambda i: (0, 0)),
        pl.BlockSpec((1, D), lambda i: (0, 0)),
    ],
    out_specs=[pl.BlockSpec((512, D), lambda i: (i, 0)),
               pl.BlockSpec((512, DP), lambda i: (i, 0)),
               pl.BlockSpec((512, DP), lambda i: (i, 0)),
               pl.BlockSpec((512, DP), lambda i: (i, 0))],
    out_shape=[jax.ShapeDtypeStruct((NP, D), jnp.float32),
               jax.ShapeDtypeStruct((NP, DP), jnp.float32),
               jax.ShapeDtypeStruct((NP, DP), jnp.float32),
               jax.ShapeDtypeStruct((NP, DP), jnp.float32)],
)


def kernel(users, pos_items, neg_items, edge_index, edge_vals,
           user_emb, item_emb,
           W1_0, b1_0, W2_0, b2_0, W1_1, b1_1, W2_1, b2_1):
    row = edge_index[0].astype(jnp.int32)
    col = edge_index[1].astype(jnp.int32)
    vals = edge_vals.astype(jnp.float32)
    ui = users.astype(jnp.int32)
    pi = pos_items.astype(jnp.int32)
    ni = neg_items.astype(jnp.int32)
    pio = pi + HALF
    nio = ni + HALF

    ego = (jnp.zeros((NP, D), jnp.float32)
           .at[:NU].set(user_emb)
           .at[HALF:HALF + NI].set(item_emb))
    tabl = jnp.concatenate([ego, jnp.zeros((NP, D), jnp.float32)], axis=1)
    tabr = jnp.concatenate([jnp.zeros((NP, D), jnp.float32), ego], axis=1)

    ue128 = jnp.concatenate(
        [user_emb, jnp.zeros((NU, D), jnp.float32)], axis=1)
    ie128 = jnp.concatenate(
        [item_emb, jnp.zeros((NI, D), jnp.float32)], axis=1)

    norms = []
    for W1, b1, W2, b2 in ((W1_0, b1_0, W2_0, b2_0),
                           (W1_1, b1_1, W2_1, b2_1)):
        tab = jnp.concatenate([tabl, tabr], axis=0)
        side = _spmm(tab, col, row, vals).reshape(NP, D)
        ego, tabl, tabr, nrm = _dense(side, ego, W1, b1, W2, b2)
        norms.append(nrm)

    (ub, pb, nb, u1, p1, n1o, u2, p2, n2o) = _gather(
        ue128, ie128, norms[0], norms[1], ui, pi, ni, pio, nio)

    u_out = jnp.concatenate([ub[:, :D], u1[:, :D], u2[:, :D]], axis=1)
    pos_out = jnp.concatenate([pb[:, :D], p1[:, :D], p2[:, :D]], axis=1)
    neg_out = jnp.concatenate([nb[:, :D], n1o[:, :D], n2o[:, :D]], axis=1)
    return (u_out, pos_out, neg_out)
